# Initial kernel scaffold; baseline (speedup 1.0000x reference)
import jax
import jax.numpy as jnp
from jax.experimental import pallas as pl


def kernel(x, coords_source):
    raise NotImplementedError("write your pallas kernel here")


# speedup vs baseline: 17.9612x; 17.9612x over previous
import functools

import numpy as np
import jax
import jax.numpy as jnp
from jax import lax
from jax.experimental import pallas as pl
from jax.experimental.pallas import tpu as pltpu
from jax.experimental.pallas import tpu_sc as plsc

B = 4
N = 16384
NF = 16
NQ = 64
CH = N // NQ
R2 = B * NQ
N2 = N // NQ
NP = 66
NI = 132
NO = 96
FT = 64

SC_NC = 2
SC_NS = 16
SC_L = 16
SC_NW = SC_NC * SC_NS
CHW = (B * N) // SC_NW


def _quantile_parts_jax(n):
    quants = jnp.linspace(1.0 / NQ, 1.0 - 1.0 / NQ, NQ - 1)
    nf = lax.convert_element_type(n, jnp.float32)
    q = lax.mul(quants, nf - 1)
    low = lax.floor(q)
    high = lax.ceil(q)
    hw = lax.sub(q, low)
    lw = lax.sub(jnp.float32(1.0), hw)
    low = lax.clamp(jnp.float32(0.0), low, nf - 1)
    high = lax.clamp(jnp.float32(0.0), high, nf - 1)
    return (low.astype(jnp.int32), high.astype(jnp.int32), lw, hw)


def _cubic_w_jax(t):
    a = -0.75
    t = jnp.abs(t)
    w1 = (a + 2.0) * t ** 3 - (a + 3.0) * t ** 2 + 1.0
    w2 = a * t ** 3 - 5 * a * t ** 2 + 8 * a * t - 4 * a
    return jnp.where(t <= 1.0, w1, jnp.where(t < 2.0, w2, jnp.zeros_like(t)))


def _tables():
    lo1, hi1, lw1, hw1 = _quantile_parts_jax(N)
    lo2, hi2, lw2, hw2 = _quantile_parts_jax(N2)
    src = jnp.linspace(0.0, float(NP - 1), NI)
    i0 = jnp.clip(jnp.floor(src).astype(jnp.int32), 0, NP - 1)
    i1 = jnp.clip(i0 + 1, 0, NP - 1)
    t = src - i0.astype(src.dtype)
    lt = 1.0 - t
    ar66 = jnp.arange(NP)
    g0 = (ar66[None, :] == i0[:, None]).astype(jnp.float32)
    g1 = (ar66[None, :] == i1[:, None]).astype(jnp.float32)
    f0 = jnp.floor(src)
    tc = src - f0
    cidx = jnp.clip(f0.astype(jnp.int32)[:, None] + jnp.arange(-1, 3)[None, :],
                    0, NP - 1)
    w = jnp.stack([_cubic_w_jax(tc + 1.0), _cubic_w_jax(tc),
                   _cubic_w_jax(1.0 - tc), _cubic_w_jax(2.0 - tc)], axis=-1)
    oo = jnp.broadcast_to(jnp.arange(NI)[:, None], (NI, 4))
    c132 = jnp.zeros((NI, NP), jnp.float32).at[oo, cidx].add(w)
    xl = jnp.linspace(0.0, 1.0, NO)
    return dict(
        pos1=jnp.stack([lo1, hi1]), lw1=lw1, hw1=hw1,
        pos2=jnp.stack([lo2, hi2]), lw2=lw2, hw2=hw2,
        g0=g0, g1=g1, c132=c132,
        bic=jnp.stack([t, lt], axis=1).astype(jnp.float32),
        bir=jnp.stack([t, lt], axis=0).astype(jnp.float32),
        xl=xl.reshape(1, NO),
    )


def _bitonic_rows(x, n, ii):
    k = 2
    while k <= n:
        j = k // 2
        while j >= 1:
            bitc = (ii & j) == 0
            nbr = jnp.where(bitc, jnp.roll(x, -j, axis=1), jnp.roll(x, j, axis=1))
            asc = (ii & k) == 0
            take_min = asc == bitc
            x = jnp.where(take_min, jnp.minimum(x, nbr), jnp.maximum(x, nbr))
            j //= 2
        k *= 2
    return x


def _k1a_body(c0_ref, pos_ref, lo_ref, hi_ref):
    c = c0_ref[...]
    rowf = lax.broadcasted_iota(jnp.int32, (B, N), 0).astype(jnp.float32)
    coff = c + rowf * np.float32(B)
    ii = lax.broadcasted_iota(jnp.int32, (B, N), 1)
    s = _bitonic_rows(coff, N, ii)
    pos = lax.broadcasted_iota(jnp.int32, (N, NQ - 1), 0)
    oh_lo = (pos == pos_ref[0:1, :]).astype(jnp.float32)
    oh_hi = (pos == pos_ref[1:2, :]).astype(jnp.float32)
    dn = (((1,), (0,)), ((), ()))
    lo_ref[...] = lax.dot_general(s, oh_lo, dn, preferred_element_type=jnp.float32, precision=lax.Precision.HIGHEST)
    hi_ref[...] = lax.dot_general(s, oh_hi, dn, preferred_element_type=jnp.float32, precision=lax.Precision.HIGHEST)


_k1a = pl.pallas_call(
    _k1a_body,
    out_shape=(jax.ShapeDtypeStruct((B, NQ - 1), jnp.float32),
               jax.ShapeDtypeStruct((B, NQ - 1), jnp.float32)),
)


def _k1b_body(c0_ref, qs_ref, perm_ref):
    c = c0_ref[...]
    rowf = lax.broadcasted_iota(jnp.int32, (B, N), 0).astype(jnp.float32)
    coff = c + rowf * np.float32(B)
    qs = qs_ref[...]
    bucket = jnp.zeros((B, N), jnp.int32)
    for k in range(NQ - 1):
        bucket = bucket + (qs[:, k:k + 1] <= coff).astype(jnp.int32)
    ii = lax.broadcasted_iota(jnp.int32, (B, N), 1)
    key = bucket * N + ii
    ks = _bitonic_rows(key, N, ii)
    rowi = lax.broadcasted_iota(jnp.int32, (B, N), 0)
    perm_ref[...] = (ks & (N - 1)) + rowi * N


_k1b = pl.pallas_call(
    _k1b_body,
    out_shape=jax.ShapeDtypeStruct((B, N), jnp.int32),
)


def _k2a_body(cs_ref, pos_ref, lo_ref, hi_ref):
    c = cs_ref[...]
    rowf = lax.broadcasted_iota(jnp.int32, (R2, N2), 0).astype(jnp.float32)
    coff = c + rowf * np.float32(R2)
    ii = lax.broadcasted_iota(jnp.int32, (R2, N2), 1)
    s = _bitonic_rows(coff, N2, ii)
    pos = lax.broadcasted_iota(jnp.int32, (N2, NQ - 1), 0)
    oh_lo = (pos == pos_ref[0:1, :]).astype(jnp.float32)
    oh_hi = (pos == pos_ref[1:2, :]).astype(jnp.float32)
    dn = (((1,), (0,)), ((), ()))
    lo_ref[...] = lax.dot_general(s, oh_lo, dn, preferred_element_type=jnp.float32, precision=lax.Precision.HIGHEST)
    hi_ref[...] = lax.dot_general(s, oh_hi, dn, preferred_element_type=jnp.float32, precision=lax.Precision.HIGHEST)


_k2a = pl.pallas_call(
    _k2a_body,
    out_shape=(jax.ShapeDtypeStruct((R2, NQ - 1), jnp.float32),
               jax.ShapeDtypeStruct((R2, NQ - 1), jnp.float32)),
)


def _k2b_body(cs_ref, qs_ref, pg_ref):
    c = cs_ref[...]
    rowf = lax.broadcasted_iota(jnp.int32, (R2, N2), 0).astype(jnp.float32)
    coff = c + rowf * np.float32(R2)
    qs = qs_ref[...]
    bucket = jnp.zeros((R2, N2), jnp.int32)
    for k in range(NQ - 1):
        bucket = bucket + (qs[:, k:k + 1] <= coff).astype(jnp.int32)
    ii = lax.broadcasted_iota(jnp.int32, (R2, N2), 1)
    key = bucket * N2 + ii
    ks = _bitonic_rows(key, N2, ii)
    rowi = lax.broadcasted_iota(jnp.int32, (R2, N2), 0)
    pg_ref[...] = (ks & (N2 - 1)) + rowi * N2


_k2b = pl.pallas_call(
    _k2b_body,
    out_shape=jax.ShapeDtypeStruct((R2, N2), jnp.int32),
)


def _kmed_body(v_ref, m_ref):
    a = v_ref[:, 0, :]
    b = v_ref[:, 1, :]
    c = v_ref[:, 2, :]
    d = v_ref[:, 3, :]
    l1 = jnp.minimum(a, b)
    h1 = jnp.maximum(a, b)
    l2 = jnp.minimum(c, d)
    h2 = jnp.maximum(c, d)
    m_ref[...] = jnp.minimum(jnp.maximum(l1, l2), jnp.minimum(h1, h2))


_kmed = pl.pallas_call(
    _kmed_body,
    out_shape=jax.ShapeDtypeStruct((2 * B, N // 4), jnp.float32),
)


def _k3_body(st_ref, cm_ref, g0_ref, g1_ref, c132_ref, bic_ref, bir_ref,
             xl_ref, out_ref):
    f32 = jnp.float32
    g0 = g0_ref[...]
    g1 = g1_ref[...]
    t_col = bic_ref[:, 0:1]
    lt_col = bic_ref[:, 1:2]
    t_row = bir_ref[0:1, :]
    lt_row = bir_ref[1:2, :]
    xl = xl_ref[...]
    dn_mm = (((1,), (0,)), ((), ()))

    cm0 = cm_ref[0]
    cm1 = cm_ref[1]
    sc0 = (cm0 - np.float32(0.0)) / np.float32(1.0)
    sc1 = (cm1 - np.float32(0.0)) / np.float32(1.0)
    mn0 = jnp.min(sc0, axis=0, keepdims=True)
    mx0 = jnp.max(sc0, axis=0, keepdims=True)
    mn1 = jnp.min(sc1, axis=1, keepdims=True)
    mx1 = jnp.max(sc1, axis=1, keepdims=True)
    sc0p = jnp.concatenate([mn0 - np.float32(0.5), sc0, mx0 + np.float32(0.5)], 0)
    sc0p = jnp.concatenate([sc0p[:, :1], sc0p, sc0p[:, -1:]], 1)
    sc1p = jnp.concatenate([mn1 - np.float32(0.5), sc1, mx1 + np.float32(0.5)], 1)
    sc1p = jnp.concatenate([sc1p[:1, :], sc1p, sc1p[-1:, :]], 0)

    def bilin(m):
        y = (lax.dot_general(g0, m, dn_mm, preferred_element_type=f32, precision=lax.Precision.HIGHEST) * lt_col
             + lax.dot_general(g1, m, dn_mm, preferred_element_type=f32, precision=lax.Precision.HIGHEST) * t_col)
        dn_r = (((1,), (1,)), ((), ()))
        z = (lax.dot_general(y, g0, dn_r, preferred_element_type=f32, precision=lax.Precision.HIGHEST) * lt_row
             + lax.dot_general(y, g1, dn_r, preferred_element_type=f32, precision=lax.Precision.HIGHEST) * t_row)
        return z

    ci0 = bilin(sc0p)
    ci1 = bilin(sc1p)

    OC = 16
    hi_io = lax.broadcasted_iota(jnp.int32, (NI, NI, OC), 0)
    cols = []
    for oc in range(0, NO, OC):
        xlc = xl[:, oc:oc + OC][:, None, :]
        dev = jnp.abs(ci0[:, :, None] - xlc)
        mn = jnp.min(dev, axis=0)
        idx = jnp.min(jnp.where(dev == mn[None], hi_io, NI), axis=0)
        cols.append(idx)
    ind0 = jnp.concatenate(cols, axis=1)

    c132 = c132_ref[...]
    st = st_ref[...]
    dn_r = (((1,), (1,)), ((), ()))
    t1 = lax.dot_general(st, c132, dn_r, preferred_element_type=f32, precision=lax.Precision.HIGHEST)
    t1r = t1.reshape(FT, NP, NI)
    dn_c = (((1,), (1,)), ((), ()))
    si = lax.dot_general(t1r, c132, dn_c, preferred_element_type=f32, precision=lax.Precision.HIGHEST)

    wi_io = lax.broadcasted_iota(jnp.int32, (NI, OC, OC), 0)
    dn_g = (((2,), (0,)), ((0,), (1,)))
    dn_g1 = (((2,), (2,)), ((0,), (1,)))
    dn_g2 = (((2,), (0,)), ((0,), (1,)))
    for oc in range(0, NO, OC):
        ind0_c = ind0[:, oc:oc + OC]
        gh_c = (ind0_c[:, :, None]
                == lax.broadcasted_iota(jnp.int32, (NI, OC, NI), 2)).astype(f32)
        ci1g_c = lax.dot_general(gh_c, ci1, dn_g, preferred_element_type=f32, precision=lax.Precision.HIGHEST)
        si1_c = lax.dot_general(gh_c, si, dn_g1, preferred_element_type=f32, precision=lax.Precision.HIGHEST)
        parts = []
        for oc2 in range(0, NO, OC):
            xlc = xl[:, oc2:oc2 + OC][:, None, :]
            dev = jnp.abs(ci1g_c[:, :, None] - xlc)
            mn = jnp.min(dev, axis=0)
            idx = jnp.min(jnp.where(dev == mn[None], wi_io, NI), axis=0)
            parts.append(idx)
        ind1_c = jnp.concatenate(parts, axis=1)
        hh_c = (ind1_c[:, :, None]
                == lax.broadcasted_iota(jnp.int32, (OC, NO, NI), 2)).astype(f32)
        out_ref[oc:oc + OC, :, :] = lax.dot_general(
            hh_c, si1_c, dn_g2, preferred_element_type=f32, precision=lax.Precision.HIGHEST)


_k3 = pl.pallas_call(
    _k3_body,
    grid=(B,),
    in_specs=[pl.BlockSpec((None, FT * NP, NP), lambda b: (b, 0, 0)),
              pl.BlockSpec((None, 2, NQ, NQ), lambda b: (b, 0, 0, 0)),
              pl.BlockSpec((NI, NP), lambda b: (0, 0)),
              pl.BlockSpec((NI, NP), lambda b: (0, 0)),
              pl.BlockSpec((NI, NP), lambda b: (0, 0)),
              pl.BlockSpec((NI, 2), lambda b: (0, 0)),
              pl.BlockSpec((2, NI), lambda b: (0, 0)),
              pl.BlockSpec((1, NO), lambda b: (0, 0))],
    out_specs=pl.BlockSpec((None, NO, NO, FT), lambda b: (b, 0, 0, 0)),
    out_shape=jax.ShapeDtypeStruct((B, NO, NO, FT), jnp.float32),
)


def _sc_wid():
    return lax.axis_index("s") * SC_NC + lax.axis_index("c")


@functools.cache
def _build_sc_gather_cs1():
    mesh = plsc.VectorSubcoreMesh(core_axis_name="c", subcore_axis_name="s",
                                  num_cores=SC_NC)

    @functools.partial(
        pl.kernel, mesh=mesh,
        compiler_params=pltpu.CompilerParams(use_tc_tiling_on_sc=False,
                                             needs_layout_passes=False),
        out_type=jax.ShapeDtypeStruct((B * N,), jnp.float32),
        scratch_types=[
            pltpu.VMEM((B * N,), jnp.float32),
            pltpu.VMEM((CHW,), jnp.int32),
            pltpu.VMEM((CHW,), jnp.float32),
        ],
    )
    def sc_gather_cs1(c1_hbm, perm_hbm, out_hbm, tab_v, idx_v, res_v):
        wid = _sc_wid()
        base = wid * CHW
        pltpu.sync_copy(c1_hbm, tab_v)
        pltpu.sync_copy(perm_hbm.at[pl.ds(base, CHW)], idx_v)

        def body(j, carry):
            sl = pl.ds(j * SC_L, SC_L)
            res_v[sl] = plsc.load_gather(tab_v, [idx_v[sl]])
            return carry

        lax.fori_loop(0, CHW // SC_L, body, 0)
        pltpu.sync_copy(res_v, out_hbm.at[pl.ds(base, CHW)])

    return sc_gather_cs1


@functools.cache
def _build_sc_gather_main():
    mesh = plsc.VectorSubcoreMesh(core_axis_name="c", subcore_axis_name="s",
                                  num_cores=SC_NC)

    @functools.partial(
        pl.kernel, mesh=mesh,
        compiler_params=pltpu.CompilerParams(use_tc_tiling_on_sc=False,
                                             needs_layout_passes=False),
        out_type=(jax.ShapeDtypeStruct((B * N, NF), jnp.float32),
                  jax.ShapeDtypeStruct((B * N,), jnp.float32),
                  jax.ShapeDtypeStruct((B * N,), jnp.float32)),
        scratch_types=[
            pltpu.VMEM((B * N,), jnp.int32),
            pltpu.VMEM((CHW,), jnp.int32),
            pltpu.VMEM((CHW,), jnp.int32),
            pltpu.VMEM((CHW,), jnp.float32),
            pltpu.VMEM((CHW,), jnp.float32),
            pltpu.VMEM((CHW, NF), jnp.float32),
            pltpu.SemaphoreType.DMA,
        ],
    )
    def sc_gather_main(perm_hbm, gp_hbm, c0i_hbm, c1i_hbm, xf_hbm,
                       data_hbm, cs20_hbm, cs21_hbm,
                       tab_v, gp_v, pt_v, a_v, b_v, rows_v, sem):
        wid = _sc_wid()
        base = wid * CHW
        nch = CHW // SC_L

        pltpu.sync_copy(perm_hbm, tab_v)
        pltpu.sync_copy(gp_hbm.at[pl.ds(base, CHW)], gp_v)

        def body1(j, carry):
            sl = pl.ds(j * SC_L, SC_L)
            pt_v[sl] = plsc.load_gather(tab_v, [gp_v[sl]])
            return carry

        lax.fori_loop(0, nch, body1, 0)

        pltpu.sync_copy(c0i_hbm, tab_v)

        def body2(j, carry):
            sl = pl.ds(j * SC_L, SC_L)
            a_v[sl] = plsc.bitcast(plsc.load_gather(tab_v, [pt_v[sl]]),
                                   jnp.float32)
            return carry

        lax.fori_loop(0, nch, body2, 0)

        pltpu.sync_copy(c1i_hbm, tab_v)

        def body3(j, carry):
            sl = pl.ds(j * SC_L, SC_L)
            b_v[sl] = plsc.bitcast(plsc.load_gather(tab_v, [pt_v[sl]]),
                                   jnp.float32)
            return carry

        lax.fori_loop(0, nch, body3, 0)

        copies = []
        for c in range(CHW // 128):
            sl = pl.ds(c * 128, 128)
            copies.append(pltpu.async_copy(xf_hbm.at[pt_v.at[sl]],
                                           rows_v.at[sl], sem))
        for cp in copies:
            cp.wait()

        pltpu.sync_copy(rows_v, data_hbm.at[pl.ds(base, CHW)])
        pltpu.sync_copy(a_v, cs20_hbm.at[pl.ds(base, CHW)])
        pltpu.sync_copy(b_v, cs21_hbm.at[pl.ds(base, CHW)])

    return sc_gather_main


def kernel(x, coords_source):
    tb = _tables()
    c0 = coords_source[:, 0, :]
    c1 = coords_source[:, 1, :]

    s_lo, s_hi = _k1a(c0, tb["pos1"])
    qs1 = lax.add(lax.mul(s_lo, tb["lw1"][None, :]),
                  lax.mul(s_hi, tb["hw1"][None, :]))
    perm1g = _k1b(c0, qs1)
    perm1gf = perm1g.reshape(-1)

    cs1 = _build_sc_gather_cs1()(c1.reshape(-1), perm1gf)
    cs1r = cs1.reshape(R2, N2)

    s2_lo, s2_hi = _k2a(cs1r, tb["pos2"])
    qs2 = lax.add(lax.mul(s2_lo, tb["lw2"][None, :]),
                  lax.mul(s2_hi, tb["hw2"][None, :]))
    pg = _k2b(cs1r, qs2)
    pgf = pg.reshape(-1)

    c0i = lax.bitcast_convert_type(c0.reshape(-1), jnp.int32)
    c1i = lax.bitcast_convert_type(c1.reshape(-1), jnp.int32)
    data, cs20, cs21 = _build_sc_gather_main()(perm1gf, pgf, c0i, c1i,
                                               x.reshape(B * N, NF))

    cs2 = jnp.stack([cs20.reshape(B, N), cs21.reshape(B, N)], axis=1)
    vmed = cs2.reshape(2 * B, N // 4, 4).transpose(0, 2, 1)
    cm = _kmed(vmed).reshape(B, 2, NQ, NQ)

    d = data.reshape(B, NQ, NQ, 4 * NF)
    d = jnp.concatenate([d[:, :, :1], d, d[:, :, -1:]], axis=2)
    d = jnp.concatenate([d[:, :1], d, d[:, -1:]], axis=1)
    st = d.transpose(0, 3, 1, 2).reshape(B, FT * NP, NP)

    out = _k3(st, cm, tb["g0"], tb["g1"], tb["c132"],
              tb["bic"], tb["bir"], tb["xl"])
    return out.transpose(0, 3, 2, 1).reshape(B, 4, NF, NO, NO)


# speedup vs baseline: 18.4611x; 1.0278x over previous
import functools

import numpy as np
import jax
import jax.numpy as jnp
from jax import lax
from jax.experimental import pallas as pl
from jax.experimental.pallas import tpu as pltpu
from jax.experimental.pallas import tpu_sc as plsc

B = 4
N = 16384
NF = 16
NQ = 64
CH = N // NQ
R2 = B * NQ
N2 = N // NQ
NP = 66
NI = 132
NO = 96
FT = 64

SC_NC = 2
SC_NS = 16
SC_L = 16
SC_NW = SC_NC * SC_NS
CHW = (B * N) // SC_NW


def _quantile_parts_jax(n):
    quants = jnp.linspace(1.0 / NQ, 1.0 - 1.0 / NQ, NQ - 1)
    nf = lax.convert_element_type(n, jnp.float32)
    q = lax.mul(quants, nf - 1)
    low = lax.floor(q)
    high = lax.ceil(q)
    hw = lax.sub(q, low)
    lw = lax.sub(jnp.float32(1.0), hw)
    low = lax.clamp(jnp.float32(0.0), low, nf - 1)
    high = lax.clamp(jnp.float32(0.0), high, nf - 1)
    return (low.astype(jnp.int32), high.astype(jnp.int32), lw, hw)


def _cubic_w_jax(t):
    a = -0.75
    t = jnp.abs(t)
    w1 = (a + 2.0) * t ** 3 - (a + 3.0) * t ** 2 + 1.0
    w2 = a * t ** 3 - 5 * a * t ** 2 + 8 * a * t - 4 * a
    return jnp.where(t <= 1.0, w1, jnp.where(t < 2.0, w2, jnp.zeros_like(t)))


def _tables():
    lo1, hi1, lw1, hw1 = _quantile_parts_jax(N)
    lo2, hi2, lw2, hw2 = _quantile_parts_jax(N2)
    src = jnp.linspace(0.0, float(NP - 1), NI)
    i0 = jnp.clip(jnp.floor(src).astype(jnp.int32), 0, NP - 1)
    i1 = jnp.clip(i0 + 1, 0, NP - 1)
    t = src - i0.astype(src.dtype)
    lt = 1.0 - t
    ar66 = jnp.arange(NP)
    g0 = (ar66[None, :] == i0[:, None]).astype(jnp.float32)
    g1 = (ar66[None, :] == i1[:, None]).astype(jnp.float32)
    f0 = jnp.floor(src)
    tc = src - f0
    cidx = jnp.clip(f0.astype(jnp.int32)[:, None] + jnp.arange(-1, 3)[None, :],
                    0, NP - 1)
    w = jnp.stack([_cubic_w_jax(tc + 1.0), _cubic_w_jax(tc),
                   _cubic_w_jax(1.0 - tc), _cubic_w_jax(2.0 - tc)], axis=-1)
    c132 = jnp.zeros((NI, NP), jnp.float32)
    for tap in range(4):
        c132 = c132 + ((ar66[None, :] == cidx[:, tap:tap + 1]).astype(jnp.float32)
                       * w[:, tap:tap + 1])
    xl = jnp.linspace(0.0, 1.0, NO)
    return dict(
        pos1=jnp.stack([lo1, hi1]), lw1=lw1, hw1=hw1,
        pos2=jnp.stack([lo2, hi2]), lw2=lw2, hw2=hw2,
        g0=g0, g1=g1, c132=c132,
        bic=jnp.stack([t, lt], axis=1).astype(jnp.float32),
        bir=jnp.stack([t, lt], axis=0).astype(jnp.float32),
        xl=xl.reshape(1, NO),
    )


def _bitonic_rows(x, n, ii):
    k = 2
    while k <= n:
        j = k // 2
        while j >= 1:
            bitc = (ii & j) == 0
            nbr = jnp.where(bitc, jnp.roll(x, -j, axis=1), jnp.roll(x, j, axis=1))
            asc = (ii & k) == 0
            take_min = asc == bitc
            x = jnp.where(take_min, jnp.minimum(x, nbr), jnp.maximum(x, nbr))
            j //= 2
        k *= 2
    return x


def _k1a_body(c0_ref, pos_ref, lo_ref, hi_ref):
    c = c0_ref[...]
    rowf = lax.broadcasted_iota(jnp.int32, (B, N), 0).astype(jnp.float32)
    coff = c + rowf * np.float32(B)
    ii = lax.broadcasted_iota(jnp.int32, (B, N), 1)
    s = _bitonic_rows(coff, N, ii)
    pos = lax.broadcasted_iota(jnp.int32, (N, NQ - 1), 0)
    oh_lo = (pos == pos_ref[0:1, :]).astype(jnp.float32)
    oh_hi = (pos == pos_ref[1:2, :]).astype(jnp.float32)
    dn = (((1,), (0,)), ((), ()))
    lo_ref[...] = lax.dot_general(s, oh_lo, dn, preferred_element_type=jnp.float32, precision=lax.Precision.HIGHEST)
    hi_ref[...] = lax.dot_general(s, oh_hi, dn, preferred_element_type=jnp.float32, precision=lax.Precision.HIGHEST)


_k1a = pl.pallas_call(
    _k1a_body,
    out_shape=(jax.ShapeDtypeStruct((B, NQ - 1), jnp.float32),
               jax.ShapeDtypeStruct((B, NQ - 1), jnp.float32)),
)


def _k1b_body(c0_ref, qs_ref, perm_ref):
    c = c0_ref[...]
    rowf = lax.broadcasted_iota(jnp.int32, (B, N), 0).astype(jnp.float32)
    coff = c + rowf * np.float32(B)
    qs = qs_ref[...]
    bucket = jnp.zeros((B, N), jnp.int32)
    for k in range(NQ - 1):
        bucket = bucket + (qs[:, k:k + 1] <= coff).astype(jnp.int32)
    ii = lax.broadcasted_iota(jnp.int32, (B, N), 1)
    key = bucket * N + ii
    ks = _bitonic_rows(key, N, ii)
    rowi = lax.broadcasted_iota(jnp.int32, (B, N), 0)
    perm_ref[...] = (ks & (N - 1)) + rowi * N


_k1b = pl.pallas_call(
    _k1b_body,
    out_shape=jax.ShapeDtypeStruct((B, N), jnp.int32),
)


def _k2a_body(cs_ref, pos_ref, lo_ref, hi_ref):
    c = cs_ref[...]
    rowf = lax.broadcasted_iota(jnp.int32, (R2, N2), 0).astype(jnp.float32)
    coff = c + rowf * np.float32(R2)
    ii = lax.broadcasted_iota(jnp.int32, (R2, N2), 1)
    s = _bitonic_rows(coff, N2, ii)
    pos = lax.broadcasted_iota(jnp.int32, (N2, NQ - 1), 0)
    oh_lo = (pos == pos_ref[0:1, :]).astype(jnp.float32)
    oh_hi = (pos == pos_ref[1:2, :]).astype(jnp.float32)
    dn = (((1,), (0,)), ((), ()))
    lo_ref[...] = lax.dot_general(s, oh_lo, dn, preferred_element_type=jnp.float32, precision=lax.Precision.HIGHEST)
    hi_ref[...] = lax.dot_general(s, oh_hi, dn, preferred_element_type=jnp.float32, precision=lax.Precision.HIGHEST)


_k2a = pl.pallas_call(
    _k2a_body,
    out_shape=(jax.ShapeDtypeStruct((R2, NQ - 1), jnp.float32),
               jax.ShapeDtypeStruct((R2, NQ - 1), jnp.float32)),
)


def _k2b_body(cs_ref, qs_ref, pg_ref):
    c = cs_ref[...]
    rowf = lax.broadcasted_iota(jnp.int32, (R2, N2), 0).astype(jnp.float32)
    coff = c + rowf * np.float32(R2)
    qs = qs_ref[...]
    bucket = jnp.zeros((R2, N2), jnp.int32)
    for k in range(NQ - 1):
        bucket = bucket + (qs[:, k:k + 1] <= coff).astype(jnp.int32)
    ii = lax.broadcasted_iota(jnp.int32, (R2, N2), 1)
    key = bucket * N2 + ii
    ks = _bitonic_rows(key, N2, ii)
    rowi = lax.broadcasted_iota(jnp.int32, (R2, N2), 0)
    pg_ref[...] = (ks & (N2 - 1)) + rowi * N2


_k2b = pl.pallas_call(
    _k2b_body,
    out_shape=jax.ShapeDtypeStruct((R2, N2), jnp.int32),
)


def _kmed_body(v_ref, m_ref):
    a = v_ref[:, 0, :]
    b = v_ref[:, 1, :]
    c = v_ref[:, 2, :]
    d = v_ref[:, 3, :]
    l1 = jnp.minimum(a, b)
    h1 = jnp.maximum(a, b)
    l2 = jnp.minimum(c, d)
    h2 = jnp.maximum(c, d)
    m_ref[...] = jnp.minimum(jnp.maximum(l1, l2), jnp.minimum(h1, h2))


_kmed = pl.pallas_call(
    _kmed_body,
    out_shape=jax.ShapeDtypeStruct((2 * B, N // 4), jnp.float32),
)


def _k3_body(st_ref, cm_ref, g0_ref, g1_ref, c132_ref, bic_ref, bir_ref,
             xl_ref, out_ref):
    f32 = jnp.float32
    g0 = g0_ref[...]
    g1 = g1_ref[...]
    t_col = bic_ref[:, 0:1]
    lt_col = bic_ref[:, 1:2]
    t_row = bir_ref[0:1, :]
    lt_row = bir_ref[1:2, :]
    xl = xl_ref[...]
    dn_mm = (((1,), (0,)), ((), ()))

    cm0 = cm_ref[0]
    cm1 = cm_ref[1]
    sc0 = (cm0 - np.float32(0.0)) / np.float32(1.0)
    sc1 = (cm1 - np.float32(0.0)) / np.float32(1.0)
    mn0 = jnp.min(sc0, axis=0, keepdims=True)
    mx0 = jnp.max(sc0, axis=0, keepdims=True)
    mn1 = jnp.min(sc1, axis=1, keepdims=True)
    mx1 = jnp.max(sc1, axis=1, keepdims=True)
    sc0p = jnp.concatenate([mn0 - np.float32(0.5), sc0, mx0 + np.float32(0.5)], 0)
    sc0p = jnp.concatenate([sc0p[:, :1], sc0p, sc0p[:, -1:]], 1)
    sc1p = jnp.concatenate([mn1 - np.float32(0.5), sc1, mx1 + np.float32(0.5)], 1)
    sc1p = jnp.concatenate([sc1p[:1, :], sc1p, sc1p[-1:, :]], 0)

    def bilin(m):
        y = (lax.dot_general(g0, m, dn_mm, preferred_element_type=f32, precision=lax.Precision.HIGHEST) * lt_col
             + lax.dot_general(g1, m, dn_mm, preferred_element_type=f32, precision=lax.Precision.HIGHEST) * t_col)
        dn_r = (((1,), (1,)), ((), ()))
        z = (lax.dot_general(y, g0, dn_r, preferred_element_type=f32, precision=lax.Precision.HIGHEST) * lt_row
             + lax.dot_general(y, g1, dn_r, preferred_element_type=f32, precision=lax.Precision.HIGHEST) * t_row)
        return z

    ci0 = bilin(sc0p)
    ci1 = bilin(sc1p)

    OC = 16
    hi_io = lax.broadcasted_iota(jnp.int32, (NI, NI, OC), 0)
    cols = []
    for oc in range(0, NO, OC):
        xlc = xl[:, oc:oc + OC][:, None, :]
        dev = jnp.abs(ci0[:, :, None] - xlc)
        mn = jnp.min(dev, axis=0)
        idx = jnp.min(jnp.where(dev == mn[None], hi_io, NI), axis=0)
        cols.append(idx)
    ind0 = jnp.concatenate(cols, axis=1)

    c132 = c132_ref[...]
    st = st_ref[...]
    dn_r = (((1,), (1,)), ((), ()))
    t1 = lax.dot_general(st, c132, dn_r, preferred_element_type=f32, precision=lax.Precision.HIGHEST)
    t1r = t1.reshape(FT, NP, NI)
    dn_c = (((1,), (1,)), ((), ()))
    si = lax.dot_general(t1r, c132, dn_c, preferred_element_type=f32, precision=lax.Precision.HIGHEST)

    wi_io = lax.broadcasted_iota(jnp.int32, (NI, OC, OC), 0)
    dn_g = (((2,), (0,)), ((0,), (1,)))
    dn_g1 = (((2,), (2,)), ((0,), (1,)))
    dn_g2 = (((2,), (0,)), ((0,), (1,)))
    for oc in range(0, NO, OC):
        ind0_c = ind0[:, oc:oc + OC]
        gh_c = (ind0_c[:, :, None]
                == lax.broadcasted_iota(jnp.int32, (NI, OC, NI), 2)).astype(f32)
        ci1g_c = lax.dot_general(gh_c, ci1, dn_g, preferred_element_type=f32, precision=lax.Precision.HIGHEST)
        si1_c = lax.dot_general(gh_c, si, dn_g1, preferred_element_type=f32, precision=lax.Precision.HIGHEST)
        parts = []
        for oc2 in range(0, NO, OC):
            xlc = xl[:, oc2:oc2 + OC][:, None, :]
            dev = jnp.abs(ci1g_c[:, :, None] - xlc)
            mn = jnp.min(dev, axis=0)
            idx = jnp.min(jnp.where(dev == mn[None], wi_io, NI), axis=0)
            parts.append(idx)
        ind1_c = jnp.concatenate(parts, axis=1)
        hh_c = (ind1_c[:, :, None]
                == lax.broadcasted_iota(jnp.int32, (OC, NO, NI), 2)).astype(f32)
        out_ref[oc:oc + OC, :, :] = lax.dot_general(
            hh_c, si1_c, dn_g2, preferred_element_type=f32, precision=lax.Precision.HIGHEST)


_k3 = pl.pallas_call(
    _k3_body,
    grid=(B,),
    in_specs=[pl.BlockSpec((None, FT * NP, NP), lambda b: (b, 0, 0)),
              pl.BlockSpec((None, 2, NQ, NQ), lambda b: (b, 0, 0, 0)),
              pl.BlockSpec((NI, NP), lambda b: (0, 0)),
              pl.BlockSpec((NI, NP), lambda b: (0, 0)),
              pl.BlockSpec((NI, NP), lambda b: (0, 0)),
              pl.BlockSpec((NI, 2), lambda b: (0, 0)),
              pl.BlockSpec((2, NI), lambda b: (0, 0)),
              pl.BlockSpec((1, NO), lambda b: (0, 0))],
    out_specs=pl.BlockSpec((None, NO, NO, FT), lambda b: (b, 0, 0, 0)),
    out_shape=jax.ShapeDtypeStruct((B, NO, NO, FT), jnp.float32),
)


def _sc_wid():
    return lax.axis_index("s") * SC_NC + lax.axis_index("c")


@functools.cache
def _build_sc_gather_cs1():
    mesh = plsc.VectorSubcoreMesh(core_axis_name="c", subcore_axis_name="s",
                                  num_cores=SC_NC)

    @functools.partial(
        pl.kernel, mesh=mesh,
        compiler_params=pltpu.CompilerParams(use_tc_tiling_on_sc=False,
                                             needs_layout_passes=False),
        out_type=jax.ShapeDtypeStruct((B * N,), jnp.float32),
        scratch_types=[
            pltpu.VMEM((B * N,), jnp.float32),
            pltpu.VMEM((CHW,), jnp.int32),
            pltpu.VMEM((CHW,), jnp.float32),
        ],
    )
    def sc_gather_cs1(c1_hbm, perm_hbm, out_hbm, tab_v, idx_v, res_v):
        wid = _sc_wid()
        base = wid * CHW
        pltpu.sync_copy(c1_hbm, tab_v)
        pltpu.sync_copy(perm_hbm.at[pl.ds(base, CHW)], idx_v)

        def body(j, carry):
            sl = pl.ds(j * SC_L, SC_L)
            res_v[sl] = plsc.load_gather(tab_v, [idx_v[sl]])
            return carry

        lax.fori_loop(0, CHW // SC_L, body, 0)
        pltpu.sync_copy(res_v, out_hbm.at[pl.ds(base, CHW)])

    return sc_gather_cs1


@functools.cache
def _build_sc_gather_main():
    mesh = plsc.VectorSubcoreMesh(core_axis_name="c", subcore_axis_name="s",
                                  num_cores=SC_NC)

    @functools.partial(
        pl.kernel, mesh=mesh,
        compiler_params=pltpu.CompilerParams(use_tc_tiling_on_sc=False,
                                             needs_layout_passes=False),
        out_type=(jax.ShapeDtypeStruct((B * N, NF), jnp.float32),
                  jax.ShapeDtypeStruct((B * N,), jnp.float32),
                  jax.ShapeDtypeStruct((B * N,), jnp.float32)),
        scratch_types=[
            pltpu.VMEM((B * N,), jnp.int32),
            pltpu.VMEM((CHW,), jnp.int32),
            pltpu.VMEM((CHW,), jnp.int32),
            pltpu.VMEM((CHW,), jnp.float32),
            pltpu.VMEM((CHW,), jnp.float32),
            pltpu.VMEM((CHW, NF), jnp.float32),
            pltpu.SemaphoreType.DMA,
        ],
    )
    def sc_gather_main(perm_hbm, gp_hbm, c0i_hbm, c1i_hbm, xf_hbm,
                       data_hbm, cs20_hbm, cs21_hbm,
                       tab_v, gp_v, pt_v, a_v, b_v, rows_v, sem):
        wid = _sc_wid()
        base = wid * CHW
        nch = CHW // SC_L

        pltpu.sync_copy(perm_hbm, tab_v)
        pltpu.sync_copy(gp_hbm.at[pl.ds(base, CHW)], gp_v)

        def body1(j, carry):
            sl = pl.ds(j * SC_L, SC_L)
            pt_v[sl] = plsc.load_gather(tab_v, [gp_v[sl]])
            return carry

        lax.fori_loop(0, nch, body1, 0)

        copies = []
        for c in range(CHW // 128):
            sl = pl.ds(c * 128, 128)
            copies.append(pltpu.async_copy(xf_hbm.at[pt_v.at[sl]],
                                           rows_v.at[sl], sem))

        pltpu.sync_copy(c0i_hbm, tab_v)

        def body2(j, carry):
            sl = pl.ds(j * SC_L, SC_L)
            a_v[sl] = plsc.bitcast(plsc.load_gather(tab_v, [pt_v[sl]]),
                                   jnp.float32)
            return carry

        lax.fori_loop(0, nch, body2, 0)

        pltpu.sync_copy(c1i_hbm, tab_v)

        def body3(j, carry):
            sl = pl.ds(j * SC_L, SC_L)
            b_v[sl] = plsc.bitcast(plsc.load_gather(tab_v, [pt_v[sl]]),
                                   jnp.float32)
            return carry

        lax.fori_loop(0, nch, body3, 0)

        for cp in copies:
            cp.wait()

        pltpu.sync_copy(rows_v, data_hbm.at[pl.ds(base, CHW)])
        pltpu.sync_copy(a_v, cs20_hbm.at[pl.ds(base, CHW)])
        pltpu.sync_copy(b_v, cs21_hbm.at[pl.ds(base, CHW)])

    return sc_gather_main


def kernel(x, coords_source):
    tb = _tables()
    c0 = coords_source[:, 0, :]
    c1 = coords_source[:, 1, :]

    s_lo, s_hi = _k1a(c0, tb["pos1"])
    qs1 = lax.add(lax.mul(s_lo, tb["lw1"][None, :]),
                  lax.mul(s_hi, tb["hw1"][None, :]))
    perm1g = _k1b(c0, qs1)
    perm1gf = perm1g.reshape(-1)

    cs1 = _build_sc_gather_cs1()(c1.reshape(-1), perm1gf)
    cs1r = cs1.reshape(R2, N2)

    s2_lo, s2_hi = _k2a(cs1r, tb["pos2"])
    qs2 = lax.add(lax.mul(s2_lo, tb["lw2"][None, :]),
                  lax.mul(s2_hi, tb["hw2"][None, :]))
    pg = _k2b(cs1r, qs2)
    pgf = pg.reshape(-1)

    c0i = lax.bitcast_convert_type(c0.reshape(-1), jnp.int32)
    c1i = lax.bitcast_convert_type(c1.reshape(-1), jnp.int32)
    data, cs20, cs21 = _build_sc_gather_main()(perm1gf, pgf, c0i, c1i,
                                               x.reshape(B * N, NF))

    cs2 = jnp.stack([cs20.reshape(B, N), cs21.reshape(B, N)], axis=1)
    vmed = cs2.reshape(2 * B, N // 4, 4).transpose(0, 2, 1)
    cm = _kmed(vmed).reshape(B, 2, NQ, NQ)

    d = data.reshape(B, NQ, NQ, 4 * NF)
    d = jnp.concatenate([d[:, :, :1], d, d[:, :, -1:]], axis=2)
    d = jnp.concatenate([d[:, :1], d, d[:, -1:]], axis=1)
    st = d.transpose(0, 3, 1, 2).reshape(B, FT * NP, NP)

    out = _k3(st, cm, tb["g0"], tb["g1"], tb["c132"],
              tb["bic"], tb["bir"], tb["xl"])
    return out.transpose(0, 3, 2, 1).reshape(B, 4, NF, NO, NO)


# speedup vs baseline: 25.4998x; 1.3813x over previous
import functools

import numpy as np
import jax
import jax.numpy as jnp
from jax import lax
from jax.experimental import pallas as pl
from jax.experimental.pallas import tpu as pltpu
from jax.experimental.pallas import tpu_sc as plsc

B = 4
N = 16384
NF = 16
NQ = 64
CH = N // NQ
R2 = B * NQ
N2 = N // NQ
NP = 66
NI = 132
NO = 96
FT = 64

SC_NC = 2
SC_NS = 16
SC_L = 16
SC_NW = SC_NC * SC_NS
CHW = (B * N) // SC_NW


def _quantile_parts_jax(n):
    quants = jnp.linspace(1.0 / NQ, 1.0 - 1.0 / NQ, NQ - 1)
    nf = lax.convert_element_type(n, jnp.float32)
    q = lax.mul(quants, nf - 1)
    low = lax.floor(q)
    high = lax.ceil(q)
    hw = lax.sub(q, low)
    lw = lax.sub(jnp.float32(1.0), hw)
    low = lax.clamp(jnp.float32(0.0), low, nf - 1)
    high = lax.clamp(jnp.float32(0.0), high, nf - 1)
    return (low.astype(jnp.int32), high.astype(jnp.int32), lw, hw)


def _cubic_w_jax(t):
    a = -0.75
    t = jnp.abs(t)
    w1 = (a + 2.0) * t ** 3 - (a + 3.0) * t ** 2 + 1.0
    w2 = a * t ** 3 - 5 * a * t ** 2 + 8 * a * t - 4 * a
    return jnp.where(t <= 1.0, w1, jnp.where(t < 2.0, w2, jnp.zeros_like(t)))


def _tables():
    lo1, hi1, lw1, hw1 = _quantile_parts_jax(N)
    lo2, hi2, lw2, hw2 = _quantile_parts_jax(N2)
    src = jnp.linspace(0.0, float(NP - 1), NI)
    i0 = jnp.clip(jnp.floor(src).astype(jnp.int32), 0, NP - 1)
    i1 = jnp.clip(i0 + 1, 0, NP - 1)
    t = src - i0.astype(src.dtype)
    lt = 1.0 - t
    ar66 = jnp.arange(NP)
    g0 = (ar66[None, :] == i0[:, None]).astype(jnp.float32)
    g1 = (ar66[None, :] == i1[:, None]).astype(jnp.float32)
    f0 = jnp.floor(src)
    tc = src - f0
    cidx = jnp.clip(f0.astype(jnp.int32)[:, None] + jnp.arange(-1, 3)[None, :],
                    0, NP - 1)
    w = jnp.stack([_cubic_w_jax(tc + 1.0), _cubic_w_jax(tc),
                   _cubic_w_jax(1.0 - tc), _cubic_w_jax(2.0 - tc)], axis=-1)
    c132 = jnp.zeros((NI, NP), jnp.float32)
    for tap in range(4):
        c132 = c132 + ((ar66[None, :] == cidx[:, tap:tap + 1]).astype(jnp.float32)
                       * w[:, tap:tap + 1])
    xl = jnp.linspace(0.0, 1.0, NO)
    return dict(
        pos1=jnp.stack([lo1, hi1]), lw1=lw1, hw1=hw1,
        pos2=jnp.stack([lo2, hi2]), lw2=lw2, hw2=hw2,
        g0=g0, g1=g1, c132=c132,
        bic=jnp.stack([t, lt], axis=1).astype(jnp.float32),
        bir=jnp.stack([t, lt], axis=0).astype(jnp.float32),
        xl=xl.reshape(1, NO),
    )


def _bitonic_rows(x, n, ii):
    k = 2
    while k <= n:
        j = k // 2
        while j >= 1:
            bitc = (ii & j) == 0
            nbr = jnp.where(bitc, jnp.roll(x, -j, axis=1), jnp.roll(x, j, axis=1))
            asc = (ii & k) == 0
            take_min = asc == bitc
            x = jnp.where(take_min, jnp.minimum(x, nbr), jnp.maximum(x, nbr))
            j //= 2
        k *= 2
    return x


def _k1a_body(c0_ref, pos_ref, lo_ref, hi_ref):
    c = c0_ref[...]
    rowf = lax.broadcasted_iota(jnp.int32, (B, N), 0).astype(jnp.float32)
    coff = c + rowf * np.float32(B)
    ii = lax.broadcasted_iota(jnp.int32, (B, N), 1)
    s = _bitonic_rows(coff, N, ii)
    pos = lax.broadcasted_iota(jnp.int32, (N, NQ - 1), 0)
    oh_lo = (pos == pos_ref[0:1, :]).astype(jnp.float32)
    oh_hi = (pos == pos_ref[1:2, :]).astype(jnp.float32)
    dn = (((1,), (0,)), ((), ()))
    lo_ref[...] = lax.dot_general(s, oh_lo, dn, preferred_element_type=jnp.float32, precision=lax.Precision.HIGHEST)
    hi_ref[...] = lax.dot_general(s, oh_hi, dn, preferred_element_type=jnp.float32, precision=lax.Precision.HIGHEST)


_k1a = pl.pallas_call(
    _k1a_body,
    out_shape=(jax.ShapeDtypeStruct((B, NQ - 1), jnp.float32),
               jax.ShapeDtypeStruct((B, NQ - 1), jnp.float32)),
)


def _k1b_body(c0_ref, qs_ref, perm_ref):
    c = c0_ref[...]
    rowf = lax.broadcasted_iota(jnp.int32, (B, N), 0).astype(jnp.float32)
    coff = c + rowf * np.float32(B)
    qs = qs_ref[...]
    bucket = jnp.zeros((B, N), jnp.int32)
    for k in range(NQ - 1):
        bucket = bucket + (qs[:, k:k + 1] <= coff).astype(jnp.int32)
    ii = lax.broadcasted_iota(jnp.int32, (B, N), 1)
    key = bucket * N + ii
    ks = _bitonic_rows(key, N, ii)
    rowi = lax.broadcasted_iota(jnp.int32, (B, N), 0)
    perm_ref[...] = (ks & (N - 1)) + rowi * N


_k1b = pl.pallas_call(
    _k1b_body,
    out_shape=jax.ShapeDtypeStruct((B, N), jnp.int32),
)


def _k2a_body(cs_ref, pos_ref, lo_ref, hi_ref):
    c = cs_ref[...]
    rowf = lax.broadcasted_iota(jnp.int32, (R2, N2), 0).astype(jnp.float32)
    coff = c + rowf * np.float32(R2)
    ii = lax.broadcasted_iota(jnp.int32, (R2, N2), 1)
    s = _bitonic_rows(coff, N2, ii)
    pos = lax.broadcasted_iota(jnp.int32, (N2, NQ - 1), 0)
    oh_lo = (pos == pos_ref[0:1, :]).astype(jnp.float32)
    oh_hi = (pos == pos_ref[1:2, :]).astype(jnp.float32)
    dn = (((1,), (0,)), ((), ()))
    lo_ref[...] = lax.dot_general(s, oh_lo, dn, preferred_element_type=jnp.float32, precision=lax.Precision.HIGHEST)
    hi_ref[...] = lax.dot_general(s, oh_hi, dn, preferred_element_type=jnp.float32, precision=lax.Precision.HIGHEST)


_k2a = pl.pallas_call(
    _k2a_body,
    out_shape=(jax.ShapeDtypeStruct((R2, NQ - 1), jnp.float32),
               jax.ShapeDtypeStruct((R2, NQ - 1), jnp.float32)),
)


def _k2b_body(cs_ref, qs_ref, pg_ref):
    c = cs_ref[...]
    rowf = lax.broadcasted_iota(jnp.int32, (R2, N2), 0).astype(jnp.float32)
    coff = c + rowf * np.float32(R2)
    qs = qs_ref[...]
    bucket = jnp.zeros((R2, N2), jnp.int32)
    for k in range(NQ - 1):
        bucket = bucket + (qs[:, k:k + 1] <= coff).astype(jnp.int32)
    ii = lax.broadcasted_iota(jnp.int32, (R2, N2), 1)
    key = bucket * N2 + ii
    ks = _bitonic_rows(key, N2, ii)
    rowi = lax.broadcasted_iota(jnp.int32, (R2, N2), 0)
    pg_ref[...] = (ks & (N2 - 1)) + rowi * N2


_k2b = pl.pallas_call(
    _k2b_body,
    out_shape=jax.ShapeDtypeStruct((R2, N2), jnp.int32),
)


def _kmed_body(v_ref, m_ref):
    a = v_ref[:, 0, :]
    b = v_ref[:, 1, :]
    c = v_ref[:, 2, :]
    d = v_ref[:, 3, :]
    l1 = jnp.minimum(a, b)
    h1 = jnp.maximum(a, b)
    l2 = jnp.minimum(c, d)
    h2 = jnp.maximum(c, d)
    m_ref[...] = jnp.minimum(jnp.maximum(l1, l2), jnp.minimum(h1, h2))


_kmed = pl.pallas_call(
    _kmed_body,
    out_shape=jax.ShapeDtypeStruct((2 * B, N // 4), jnp.float32),
)


def _k3_body(sp_ref, cm_ref, g0_ref, g1_ref, c132_ref, bic_ref, bir_ref,
             xl_ref, si_ref, fidx_ref):
    f32 = jnp.float32
    g0 = g0_ref[...]
    g1 = g1_ref[...]
    t_col = bic_ref[:, 0:1]
    lt_col = bic_ref[:, 1:2]
    t_row = bir_ref[0:1, :]
    lt_row = bir_ref[1:2, :]
    xl = xl_ref[...]
    dn_mm = (((1,), (0,)), ((), ()))

    cm0 = cm_ref[0]
    cm1 = cm_ref[1]
    sc0 = (cm0 - np.float32(0.0)) / np.float32(1.0)
    sc1 = (cm1 - np.float32(0.0)) / np.float32(1.0)
    mn0 = jnp.min(sc0, axis=0, keepdims=True)
    mx0 = jnp.max(sc0, axis=0, keepdims=True)
    mn1 = jnp.min(sc1, axis=1, keepdims=True)
    mx1 = jnp.max(sc1, axis=1, keepdims=True)
    sc0p = jnp.concatenate([mn0 - np.float32(0.5), sc0, mx0 + np.float32(0.5)], 0)
    sc0p = jnp.concatenate([sc0p[:, :1], sc0p, sc0p[:, -1:]], 1)
    sc1p = jnp.concatenate([mn1 - np.float32(0.5), sc1, mx1 + np.float32(0.5)], 1)
    sc1p = jnp.concatenate([sc1p[:1, :], sc1p, sc1p[-1:, :]], 0)

    def bilin(m):
        y = (lax.dot_general(g0, m, dn_mm, preferred_element_type=f32, precision=lax.Precision.HIGHEST) * lt_col
             + lax.dot_general(g1, m, dn_mm, preferred_element_type=f32, precision=lax.Precision.HIGHEST) * t_col)
        dn_r = (((1,), (1,)), ((), ()))
        z = (lax.dot_general(y, g0, dn_r, preferred_element_type=f32, precision=lax.Precision.HIGHEST) * lt_row
             + lax.dot_general(y, g1, dn_r, preferred_element_type=f32, precision=lax.Precision.HIGHEST) * t_row)
        return z

    ci0 = bilin(sc0p)
    ci1 = bilin(sc1p)

    OC = 16
    hi_io = lax.broadcasted_iota(jnp.int32, (NI, NI, OC), 0)
    cols = []
    for oc in range(0, NO, OC):
        xlc = xl[:, oc:oc + OC][:, None, :]
        dev = jnp.abs(ci0[:, :, None] - xlc)
        mn = jnp.min(dev, axis=0)
        idx = jnp.min(jnp.where(dev == mn[None], hi_io, NI), axis=0)
        cols.append(idx)
    ind0 = jnp.concatenate(cols, axis=1)

    c132 = c132_ref[...]
    sp = sp_ref[...]
    dn_u = (((1,), (1,)), ((), ()))
    u = lax.dot_general(c132, sp, dn_u, preferred_element_type=f32,
                        precision=lax.Precision.HIGHEST)
    si = lax.dot_general(c132, u, dn_u, preferred_element_type=f32,
                         precision=lax.Precision.HIGHEST)
    si_ref[...] = si

    bofs = pl.program_id(0) * (NI * NI)
    wi_io = lax.broadcasted_iota(jnp.int32, (NI, OC, OC), 0)
    for oc in range(0, NO, OC):
        ind0_c = ind0[:, oc:oc + OC]
        mask_h = (lax.broadcasted_iota(jnp.int32, (NI, NI, OC), 0)
                  == ind0_c[None, :, :])
        ci1g_c = jnp.sum(jnp.where(mask_h, ci1[:, :, None], np.float32(0.0)),
                         axis=0)
        parts = []
        for oc2 in range(0, NO, OC):
            xlc = xl[:, oc2:oc2 + OC][:, None, :]
            dev = jnp.abs(ci1g_c[:, :, None] - xlc)
            mn = jnp.min(dev, axis=0)
            idx = jnp.min(jnp.where(dev == mn[None], wi_io, NI), axis=0)
            parts.append(idx)
        ind1_c = jnp.concatenate(parts, axis=1)
        mask_w = (lax.broadcasted_iota(jnp.int32, (NI, OC, NO), 0)
                  == ind1_c[None, :, :])
        hstar = jnp.sum(jnp.where(mask_w, ind0_c.astype(f32)[:, :, None],
                                  np.float32(0.0)), axis=0).astype(jnp.int32)
        fidx_ref[oc:oc + OC, :] = bofs + hstar * NI + ind1_c


_k3 = pl.pallas_call(
    _k3_body,
    grid=(B,),
    in_specs=[pl.BlockSpec((None, NP, NP, FT), lambda b: (b, 0, 0, 0)),
              pl.BlockSpec((None, 2, NQ, NQ), lambda b: (b, 0, 0, 0)),
              pl.BlockSpec((NI, NP), lambda b: (0, 0)),
              pl.BlockSpec((NI, NP), lambda b: (0, 0)),
              pl.BlockSpec((NI, NP), lambda b: (0, 0)),
              pl.BlockSpec((NI, 2), lambda b: (0, 0)),
              pl.BlockSpec((2, NI), lambda b: (0, 0)),
              pl.BlockSpec((1, NO), lambda b: (0, 0))],
    out_specs=[pl.BlockSpec((None, NI, NI, FT), lambda b: (b, 0, 0, 0)),
               pl.BlockSpec((None, NO, NO), lambda b: (b, 0, 0))],
    out_shape=(jax.ShapeDtypeStruct((B, NI, NI, FT), jnp.float32),
               jax.ShapeDtypeStruct((B, NO, NO), jnp.int32)),
)


def _sc_wid():
    return lax.axis_index("s") * SC_NC + lax.axis_index("c")


@functools.cache
def _build_sc_gather_cs1():
    mesh = plsc.VectorSubcoreMesh(core_axis_name="c", subcore_axis_name="s",
                                  num_cores=SC_NC)

    @functools.partial(
        pl.kernel, mesh=mesh,
        compiler_params=pltpu.CompilerParams(use_tc_tiling_on_sc=False,
                                             needs_layout_passes=False),
        out_type=jax.ShapeDtypeStruct((B * N,), jnp.float32),
        scratch_types=[
            pltpu.VMEM((B * N,), jnp.float32),
            pltpu.VMEM((CHW,), jnp.int32),
            pltpu.VMEM((CHW,), jnp.float32),
        ],
    )
    def sc_gather_cs1(c1_hbm, perm_hbm, out_hbm, tab_v, idx_v, res_v):
        wid = _sc_wid()
        base = wid * CHW
        pltpu.sync_copy(c1_hbm, tab_v)
        pltpu.sync_copy(perm_hbm.at[pl.ds(base, CHW)], idx_v)

        def body(j, carry):
            sl = pl.ds(j * SC_L, SC_L)
            res_v[sl] = plsc.load_gather(tab_v, [idx_v[sl]])
            return carry

        lax.fori_loop(0, CHW // SC_L, body, 0)
        pltpu.sync_copy(res_v, out_hbm.at[pl.ds(base, CHW)])

    return sc_gather_cs1


@functools.cache
def _build_sc_gather_main():
    mesh = plsc.VectorSubcoreMesh(core_axis_name="c", subcore_axis_name="s",
                                  num_cores=SC_NC)

    @functools.partial(
        pl.kernel, mesh=mesh,
        compiler_params=pltpu.CompilerParams(use_tc_tiling_on_sc=False,
                                             needs_layout_passes=False),
        out_type=(jax.ShapeDtypeStruct((B * N, NF), jnp.float32),
                  jax.ShapeDtypeStruct((B * N,), jnp.float32),
                  jax.ShapeDtypeStruct((B * N,), jnp.float32)),
        scratch_types=[
            pltpu.VMEM((B * N,), jnp.int32),
            pltpu.VMEM((CHW,), jnp.int32),
            pltpu.VMEM((CHW,), jnp.int32),
            pltpu.VMEM((CHW,), jnp.float32),
            pltpu.VMEM((CHW,), jnp.float32),
            pltpu.VMEM((CHW, NF), jnp.float32),
            pltpu.SemaphoreType.DMA,
        ],
    )
    def sc_gather_main(perm_hbm, gp_hbm, c0i_hbm, c1i_hbm, xf_hbm,
                       data_hbm, cs20_hbm, cs21_hbm,
                       tab_v, gp_v, pt_v, a_v, b_v, rows_v, sem):
        wid = _sc_wid()
        base = wid * CHW
        nch = CHW // SC_L

        pltpu.sync_copy(perm_hbm, tab_v)
        pltpu.sync_copy(gp_hbm.at[pl.ds(base, CHW)], gp_v)

        def body1(j, carry):
            sl = pl.ds(j * SC_L, SC_L)
            pt_v[sl] = plsc.load_gather(tab_v, [gp_v[sl]])
            return carry

        lax.fori_loop(0, nch, body1, 0)

        copies = []
        for c in range(CHW // 128):
            sl = pl.ds(c * 128, 128)
            copies.append(pltpu.async_copy(xf_hbm.at[pt_v.at[sl]],
                                           rows_v.at[sl], sem))

        pltpu.sync_copy(c0i_hbm, tab_v)

        def body2(j, carry):
            sl = pl.ds(j * SC_L, SC_L)
            a_v[sl] = plsc.bitcast(plsc.load_gather(tab_v, [pt_v[sl]]),
                                   jnp.float32)
            return carry

        lax.fori_loop(0, nch, body2, 0)

        pltpu.sync_copy(c1i_hbm, tab_v)

        def body3(j, carry):
            sl = pl.ds(j * SC_L, SC_L)
            b_v[sl] = plsc.bitcast(plsc.load_gather(tab_v, [pt_v[sl]]),
                                   jnp.float32)
            return carry

        lax.fori_loop(0, nch, body3, 0)

        for cp in copies:
            cp.wait()

        pltpu.sync_copy(rows_v, data_hbm.at[pl.ds(base, CHW)])
        pltpu.sync_copy(a_v, cs20_hbm.at[pl.ds(base, CHW)])
        pltpu.sync_copy(b_v, cs21_hbm.at[pl.ds(base, CHW)])

    return sc_gather_main


CHF = (B * NO * NO) // SC_NW


@functools.cache
def _build_sc_gather_field():
    mesh = plsc.VectorSubcoreMesh(core_axis_name="c", subcore_axis_name="s",
                                  num_cores=SC_NC)

    @functools.partial(
        pl.kernel, mesh=mesh,
        compiler_params=pltpu.CompilerParams(use_tc_tiling_on_sc=False,
                                             needs_layout_passes=False),
        out_type=jax.ShapeDtypeStruct((B * NO * NO, FT), jnp.float32),
        scratch_types=[
            pltpu.VMEM((CHF,), jnp.int32),
            pltpu.VMEM((CHF, FT), jnp.float32),
            pltpu.SemaphoreType.DMA,
        ],
    )
    def sc_gather_field(si_hbm, idx_hbm, out_hbm, idx_v, rows_v, sem):
        wid = _sc_wid()
        base = wid * CHF
        pltpu.sync_copy(idx_hbm.at[pl.ds(base, CHF)], idx_v)
        copies = []
        for c in range(CHF // 128):
            sl = pl.ds(c * 128, 128)
            copies.append(pltpu.async_copy(si_hbm.at[idx_v.at[sl]],
                                           rows_v.at[sl], sem))
        for cp in copies:
            cp.wait()
        pltpu.sync_copy(rows_v, out_hbm.at[pl.ds(base, CHF)])

    return sc_gather_field


def kernel(x, coords_source):
    tb = _tables()
    c0 = coords_source[:, 0, :]
    c1 = coords_source[:, 1, :]

    s_lo, s_hi = _k1a(c0, tb["pos1"])
    qs1 = lax.add(lax.mul(s_lo, tb["lw1"][None, :]),
                  lax.mul(s_hi, tb["hw1"][None, :]))
    perm1g = _k1b(c0, qs1)
    perm1gf = perm1g.reshape(-1)

    cs1 = _build_sc_gather_cs1()(c1.reshape(-1), perm1gf)
    cs1r = cs1.reshape(R2, N2)

    s2_lo, s2_hi = _k2a(cs1r, tb["pos2"])
    qs2 = lax.add(lax.mul(s2_lo, tb["lw2"][None, :]),
                  lax.mul(s2_hi, tb["hw2"][None, :]))
    pg = _k2b(cs1r, qs2)
    pgf = pg.reshape(-1)

    c0i = lax.bitcast_convert_type(c0.reshape(-1), jnp.int32)
    c1i = lax.bitcast_convert_type(c1.reshape(-1), jnp.int32)
    data, cs20, cs21 = _build_sc_gather_main()(perm1gf, pgf, c0i, c1i,
                                               x.reshape(B * N, NF))

    cs2 = jnp.stack([cs20.reshape(B, N), cs21.reshape(B, N)], axis=1)
    vmed = cs2.reshape(2 * B, N // 4, 4).transpose(0, 2, 1)
    cm = _kmed(vmed).reshape(B, 2, NQ, NQ)

    d = data.reshape(B, NQ, NQ, 4 * NF)
    d = jnp.concatenate([d[:, :, :1], d, d[:, :, -1:]], axis=2)
    d = jnp.concatenate([d[:, :1], d, d[:, -1:]], axis=1)

    si, fidx = _k3(d, cm, tb["g0"], tb["g1"], tb["c132"],
                   tb["bic"], tb["bir"], tb["xl"])
    rows = _build_sc_gather_field()(si.reshape(B * NI * NI, FT),
                                    fidx.reshape(-1))
    out = rows.reshape(B, NO, NO, FT)
    return out.transpose(0, 3, 2, 1).reshape(B, 4, NF, NO, NO)
